# Initial kernel scaffold; baseline (speedup 1.0000x reference)
import jax
import jax.numpy as jnp
from jax.experimental import pallas as pl


def kernel(pos_ids, embedding_table):
    raise NotImplementedError("write your pallas kernel here")


# speedup vs baseline: 9.1630x; 9.1630x over previous
import functools

import jax
import jax.numpy as jnp
from jax import lax
from jax.experimental import pallas as pl
from jax.experimental.pallas import tpu as pltpu
from jax.experimental.pallas import tpu_sc as plsc

NC = 2
NS = 16
NW = NC * NS
CH = 128
NBUF = 4


@functools.lru_cache(maxsize=None)
def _make_gather(NJ: int, D: int):
    B = NW * NJ * CH
    mesh = plsc.VectorSubcoreMesh(
        core_axis_name="c", subcore_axis_name="s",
        num_cores=NC, num_subcores=NS,
    )

    @functools.partial(
        pl.kernel,
        out_type=jax.ShapeDtypeStruct((B, D), jnp.float32),
        mesh=mesh,
        scratch_types=[
            pltpu.VMEM((NJ, CH), jnp.int32),
            pltpu.VMEM((NBUF, CH, D), jnp.float32),
            pltpu.SemaphoreType.DMA((NBUF,)),
            pltpu.SemaphoreType.DMA((NBUF,)),
        ],
    )
    def gather_kernel(idx_hbm, table_hbm, out_hbm, idx_v, rows_v, gsem, ssem):
        wid = lax.axis_index("s") * NC + lax.axis_index("c")
        rbase = wid * (NJ * CH)

        pltpu.sync_copy(idx_hbm.at[wid], idx_v)

        def start_gather(j, b):
            pltpu.async_copy(table_hbm.at[idx_v.at[j]], rows_v.at[b],
                             gsem.at[b])

        def wait_gather(b):
            pltpu.make_async_copy(table_hbm.at[idx_v.at[0]], rows_v.at[b],
                                  gsem.at[b]).wait()

        def start_store(j, b):
            pltpu.async_copy(rows_v.at[b],
                             out_hbm.at[pl.ds(rbase + j * CH, CH)],
                             ssem.at[b])

        def wait_store(b):
            pltpu.make_async_copy(rows_v.at[b],
                                  out_hbm.at[pl.ds(rbase, CH)],
                                  ssem.at[b]).wait()

        for b in range(NBUF):
            start_gather(b, b)

        def body(g, carry):
            j0 = g * NBUF
            for b in range(NBUF):
                wait_gather(b)
                start_store(j0 + b, b)
            for b in range(NBUF):
                wait_store(b)
                start_gather(j0 + NBUF + b, b)
            return carry

        lax.fori_loop(0, NJ // NBUF - 1, body, 0, unroll=False)

        j0 = NJ - NBUF
        for b in range(NBUF):
            wait_gather(b)
            start_store(j0 + b, b)
        for b in range(NBUF):
            wait_store(b)

    return gather_kernel


def kernel(pos_ids, embedding_table):
    Bb, S = pos_ids.shape
    V, D = embedding_table.shape
    n = Bb * S
    NJ = n // (NW * CH)
    idx = pos_ids.reshape(NW, NJ, CH).astype(jnp.int32)
    out = _make_gather(NJ, D)(idx, embedding_table)
    return out.reshape(Bb, S, D)
